# Initial kernel scaffold; baseline (speedup 1.0000x reference)
import jax
import jax.numpy as jnp
from jax.experimental import pallas as pl


def kernel(input, edge_index, edge_weight):
    raise NotImplementedError("write your pallas kernel here")


# speedup vs baseline: 4.5692x; 4.5692x over previous
import functools

import jax
import jax.numpy as jnp
from jax import lax
from jax.experimental import pallas as pl
from jax.experimental.pallas import tpu as pltpu
from jax.experimental.pallas import tpu_sc as plsc

N_NODES = 10000
D = 128
E = 320000
LANES = 16
NC = 2
NS = 16
NW = NC * NS
N_PAD = 10240
EDGES_PER_W = E // NW
BLK = 80
NBLK = EDGES_PER_W // BLK
ROWS_PER_TILE = N_PAD // NS
ZROWS = 128

_GDN = lax.GatherDimensionNumbers(
    offset_dims=(), collapsed_slice_dims=(0,), start_index_map=(0,))


def _bcast_lane(v16, lane):
    idx = jnp.full((LANES, 1), lane, dtype=jnp.int32)
    return lax.gather(v16, idx, _GDN, (1,),
                      mode=lax.GatherScatterMode.PROMISE_IN_BOUNDS)


def _sc_body(x_hbm, src_hbm, dst_hbm, w_hbm, out_hbm,
             acc, zb, src_v, dst_v, w_v, rows_v, sem):
    c = lax.axis_index("c")
    s = lax.axis_index("s")
    wid = s * NC + c

    def zrow(i, carry):
        for j in range(D // LANES):
            zb[i, pl.ds(j * LANES, LANES)] = jnp.zeros((LANES,), jnp.float32)
        return carry

    lax.fori_loop(0, ZROWS, zrow, 0)
    for k in range(ROWS_PER_TILE // ZROWS):
        pltpu.sync_copy(zb, acc.at[pl.ds(s * ROWS_PER_TILE + k * ZROWS, ZROWS)])
    plsc.subcore_barrier()

    base_w = wid * EDGES_PER_W

    def blk(i, carry):
        base = base_w + i * BLK
        pltpu.sync_copy(src_hbm.at[pl.ds(base, BLK)], src_v)
        pltpu.sync_copy(dst_hbm.at[pl.ds(base, BLK)], dst_v)
        pltpu.sync_copy(w_hbm.at[pl.ds(base, BLK)], w_v)
        pltpu.async_copy(x_hbm.at[src_v], rows_v, sem).wait()

        def grp(g, gcarry):
            w16 = w_v[pl.ds(g * LANES, LANES)]
            for l in range(LANES):
                bc = _bcast_lane(w16, l)
                e = g * LANES + l
                for j in range(D // LANES):
                    rows_v[e, pl.ds(j * LANES, LANES)] = (
                        rows_v[e, pl.ds(j * LANES, LANES)] * bc)
            return gcarry

        lax.fori_loop(0, BLK // LANES, grp, 0)
        pltpu.sync_copy(rows_v, acc.at[dst_v], add=True)
        return carry

    lax.fori_loop(0, NBLK, blk, 0)
    plsc.subcore_barrier()

    pltpu.sync_copy(acc.at[pl.ds(s * ROWS_PER_TILE, ROWS_PER_TILE)],
                    out_hbm.at[c, pl.ds(s * ROWS_PER_TILE, ROWS_PER_TILE)])


_sc_call = pl.kernel(
    _sc_body,
    out_type=jax.ShapeDtypeStruct((NC, N_PAD, D), jnp.float32),
    mesh=plsc.VectorSubcoreMesh(core_axis_name="c", subcore_axis_name="s"),
    scratch_types=[
        pltpu.VMEM_SHARED((N_PAD, D), jnp.float32),
        pltpu.VMEM((ZROWS, D), jnp.float32),
        pltpu.VMEM((BLK,), jnp.int32),
        pltpu.VMEM((BLK,), jnp.int32),
        pltpu.VMEM((BLK,), jnp.float32),
        pltpu.VMEM((BLK, D), jnp.float32),
        pltpu.SemaphoreType.DMA,
    ],
    name="gcn_spmm_sc",
)

_CBLK = 2000


def _combine_body(p_ref, q_ref, o_ref):
    o_ref[...] = p_ref[0] + q_ref[0]


_combine = pl.pallas_call(
    _combine_body,
    grid=(N_NODES // _CBLK,),
    in_specs=[
        pl.BlockSpec((1, _CBLK, D), lambda i: (0, i, 0)),
        pl.BlockSpec((1, _CBLK, D), lambda i: (1, i, 0)),
    ],
    out_specs=pl.BlockSpec((_CBLK, D), lambda i: (i, 0)),
    out_shape=jax.ShapeDtypeStruct((N_NODES, D), jnp.float32),
)


def kernel(input, edge_index, edge_weight):
    src = edge_index[0]
    dst = edge_index[1]
    partials = _sc_call(input, src, dst, edge_weight)
    return _combine(partials, partials)


# speedup vs baseline: 11.4391x; 2.5035x over previous
import jax
import jax.numpy as jnp
from jax import lax
from jax.experimental import pallas as pl
from jax.experimental.pallas import tpu as pltpu
from jax.experimental.pallas import tpu_sc as plsc

N_NODES = 10000
D = 128
E = 320000
LANES = 16
NC = 2
NS = 16
NW = NC * NS
N_PAD = 10240
EDGES_PER_W = E // NW
BLK = 80
NBLK = EDGES_PER_W // BLK
ROWS_PER_TILE = N_PAD // NS

_GDN = lax.GatherDimensionNumbers(
    offset_dims=(), collapsed_slice_dims=(0,), start_index_map=(0,))


def _bcast_lane(v16, lane):
    idx = jnp.full((LANES, 1), lane, dtype=jnp.int32)
    return lax.gather(v16, idx, _GDN, (1,),
                      mode=lax.GatherScatterMode.PROMISE_IN_BOUNDS)


def _sc_body(x_hbm, src_hbm, dst_hbm, w_hbm, out_hbm,
             acc, src_all, w_all, dst_a, dst_b, buf_a, buf_b,
             sem_a, sem_b, sem_da, sem_db):
    c = lax.axis_index("c")
    s = lax.axis_index("s")
    wid = s * NC + c
    ebase = wid * EDGES_PER_W

    pltpu.sync_copy(src_hbm.at[pl.ds(ebase, EDGES_PER_W)], src_all)
    pltpu.sync_copy(w_hbm.at[pl.ds(ebase, EDGES_PER_W)], w_all)

    def zrow(i, carry):
        for j in range(D // LANES):
            buf_a[i, pl.ds(j * LANES, LANES)] = jnp.zeros((LANES,), jnp.float32)
        return carry

    lax.fori_loop(0, BLK, zrow, 0)
    for k in range(ROWS_PER_TILE // BLK):
        pltpu.sync_copy(buf_a, acc.at[pl.ds(s * ROWS_PER_TILE + k * BLK, BLK)])
    plsc.subcore_barrier()

    def scale(buf, i):
        def grp(g, gcarry):
            w16 = w_all[pl.ds(i * BLK + g * LANES, LANES)]
            for l in range(LANES):
                bc = _bcast_lane(w16, l)
                e = g * LANES + l
                for j in range(D // LANES):
                    buf[e, pl.ds(j * LANES, LANES)] = (
                        buf[e, pl.ds(j * LANES, LANES)] * bc)
            return gcarry

        lax.fori_loop(0, BLK // LANES, grp, 0)

    def gather(i, buf, sem, dbuf, dsem):
        pltpu.async_copy(x_hbm.at[src_all.at[pl.ds(i * BLK, BLK)]], buf, sem)
        pltpu.async_copy(dst_hbm.at[pl.ds(ebase + i * BLK, BLK)], dbuf, dsem)

    def wait(i, buf, sem, dbuf, dsem):
        pltpu.make_async_copy(
            x_hbm.at[src_all.at[pl.ds(i * BLK, BLK)]], buf, sem).wait()
        pltpu.make_async_copy(
            dst_hbm.at[pl.ds(ebase + i * BLK, BLK)], dbuf, dsem).wait()

    def scatter(dbuf, buf):
        pltpu.sync_copy(buf, acc.at[dbuf], add=True)

    gather(0, buf_a, sem_a, dst_a, sem_da)
    wait(0, buf_a, sem_a, dst_a, sem_da)
    scale(buf_a, 0)
    scatter(dst_a, buf_a)
    gather(1, buf_a, sem_a, dst_a, sem_da)

    def pair(k, carry):
        i = 1 + 2 * k
        gather(i + 1, buf_b, sem_b, dst_b, sem_db)
        wait(i, buf_a, sem_a, dst_a, sem_da)
        scale(buf_a, i)
        scatter(dst_a, buf_a)

        @pl.when(i + 2 < NBLK)
        def _():
            gather(i + 2, buf_a, sem_a, dst_a, sem_da)

        wait(i + 1, buf_b, sem_b, dst_b, sem_db)
        scale(buf_b, i + 1)
        scatter(dst_b, buf_b)
        return carry

    lax.fori_loop(0, (NBLK - 1) // 2, pair, 0)
    plsc.subcore_barrier()

    pltpu.sync_copy(acc.at[pl.ds(s * ROWS_PER_TILE, ROWS_PER_TILE)],
                    out_hbm.at[c, pl.ds(s * ROWS_PER_TILE, ROWS_PER_TILE)])


_sc_call = pl.kernel(
    _sc_body,
    out_type=jax.ShapeDtypeStruct((NC, N_PAD, D), jnp.float32),
    mesh=plsc.VectorSubcoreMesh(core_axis_name="c", subcore_axis_name="s"),
    scratch_types=[
        pltpu.VMEM_SHARED((N_PAD, D), jnp.float32),
        pltpu.VMEM((EDGES_PER_W,), jnp.int32),
        pltpu.VMEM((EDGES_PER_W,), jnp.float32),
        pltpu.VMEM((BLK,), jnp.int32),
        pltpu.VMEM((BLK,), jnp.int32),
        pltpu.VMEM((BLK, D), jnp.float32),
        pltpu.VMEM((BLK, D), jnp.float32),
        pltpu.SemaphoreType.DMA,
        pltpu.SemaphoreType.DMA,
        pltpu.SemaphoreType.DMA,
        pltpu.SemaphoreType.DMA,
    ],
    name="gcn_spmm_sc",
)

_CBLK = 2000


def _combine_body(p_ref, q_ref, o_ref):
    o_ref[...] = p_ref[0] + q_ref[0]


_combine = pl.pallas_call(
    _combine_body,
    grid=(N_NODES // _CBLK,),
    in_specs=[
        pl.BlockSpec((1, _CBLK, D), lambda i: (0, i, 0)),
        pl.BlockSpec((1, _CBLK, D), lambda i: (1, i, 0)),
    ],
    out_specs=pl.BlockSpec((_CBLK, D), lambda i: (i, 0)),
    out_shape=jax.ShapeDtypeStruct((N_NODES, D), jnp.float32),
)


def kernel(input, edge_index, edge_weight):
    src = edge_index[0]
    dst = edge_index[1]
    partials = _sc_call(input, src, dst, edge_weight)
    return _combine(partials, partials)


# speedup vs baseline: 12.9076x; 1.1284x over previous
import jax
import jax.numpy as jnp
from jax import lax
from jax.experimental import pallas as pl
from jax.experimental.pallas import tpu as pltpu
from jax.experimental.pallas import tpu_sc as plsc

N_NODES = 10000
D = 128
E = 320000
LANES = 16
NC = 2
NS = 16
NW = NC * NS
N_PAD = 10240
EDGES_PER_W = E // NW
BLK = 80
NBLK = EDGES_PER_W // BLK
ROWS_PER_TILE = N_PAD // NS

_GDN = lax.GatherDimensionNumbers(
    offset_dims=(), collapsed_slice_dims=(0,), start_index_map=(0,))


def _bcast_lane(v16, lane):
    idx = jnp.full((LANES, 1), lane, dtype=jnp.int32)
    return lax.gather(v16, idx, _GDN, (1,),
                      mode=lax.GatherScatterMode.PROMISE_IN_BOUNDS)


def _sc_body(x_hbm, src_hbm, dst_hbm, w_hbm, out_hbm, acc, src_all,
             dst_0, dst_1, dst_2, w_0, w_1, w_2, buf_0, buf_1, buf_2,
             sg_0, sg_1, sg_2, ss_0, ss_1, ss_2):
    c = lax.axis_index("c")
    s = lax.axis_index("s")
    wid = s * NC + c
    ebase = wid * EDGES_PER_W

    dsts = (dst_0, dst_1, dst_2)
    ws = (w_0, w_1, w_2)
    bufs = (buf_0, buf_1, buf_2)
    sgs = (sg_0, sg_1, sg_2)
    sss = (ss_0, ss_1, ss_2)

    pltpu.sync_copy(src_hbm.at[pl.ds(ebase, EDGES_PER_W)], src_all)

    def zrow(i, carry):
        for j in range(D // LANES):
            buf_0[i, pl.ds(j * LANES, LANES)] = jnp.zeros((LANES,), jnp.float32)
        return carry

    lax.fori_loop(0, BLK, zrow, 0)
    for k in range(ROWS_PER_TILE // BLK):
        pltpu.sync_copy(buf_0, acc.at[pl.ds(s * ROWS_PER_TILE + k * BLK, BLK)])
    plsc.subcore_barrier()

    def issue(i, t):
        pltpu.async_copy(x_hbm.at[src_all.at[pl.ds(i * BLK, BLK)]],
                         bufs[t], sgs[t])
        pltpu.async_copy(dst_hbm.at[pl.ds(ebase + i * BLK, BLK)],
                         dsts[t], sgs[t])
        pltpu.async_copy(w_hbm.at[pl.ds(ebase + i * BLK, BLK)],
                         ws[t], sgs[t])

    def wait_in(i, t):
        pltpu.make_async_copy(
            x_hbm.at[src_all.at[pl.ds(i * BLK, BLK)]], bufs[t], sgs[t]).wait()
        pltpu.make_async_copy(
            dst_hbm.at[pl.ds(ebase + i * BLK, BLK)], dsts[t], sgs[t]).wait()
        pltpu.make_async_copy(
            w_hbm.at[pl.ds(ebase + i * BLK, BLK)], ws[t], sgs[t]).wait()

    def scale(i, t):
        buf, wref = bufs[t], ws[t]

        def grp(g, gcarry):
            w16 = wref[pl.ds(g * LANES, LANES)]
            for l in range(LANES):
                bc = _bcast_lane(w16, l)
                e = g * LANES + l
                for j in range(D // LANES):
                    buf[e, pl.ds(j * LANES, LANES)] = (
                        buf[e, pl.ds(j * LANES, LANES)] * bc)
            return gcarry

        lax.fori_loop(0, BLK // LANES, grp, 0)

    def process(i, t):
        wait_in(i, t)
        scale(i, t)
        pltpu.async_copy(bufs[t], acc.at[dsts[t]], sss[t], add=True)

    def wait_sc(t):
        pltpu.make_async_copy(bufs[t], acc.at[dsts[t]], sss[t]).wait()

    issue(0, 0)
    issue(1, 1)
    process(0, 0)
    issue(2, 2)
    process(1, 1)
    wait_sc(0)
    issue(3, 0)
    process(2, 2)
    wait_sc(1)
    issue(4, 1)

    def triple(k, carry):
        i = 3 * k
        wait_sc(2)
        issue(i + 2, 2)
        process(i, 0)
        wait_sc(0)
        issue(i + 3, 0)
        process(i + 1, 1)
        wait_sc(1)
        issue(i + 4, 1)
        process(i + 2, 2)
        return carry

    lax.fori_loop(1, (NBLK - 2) // 3, triple, 0)
    process(NBLK - 2, 0)
    process(NBLK - 1, 1)
    wait_sc(2)
    wait_sc(0)
    wait_sc(1)
    plsc.subcore_barrier()

    pltpu.sync_copy(acc.at[pl.ds(s * ROWS_PER_TILE, ROWS_PER_TILE)],
                    out_hbm.at[c, pl.ds(s * ROWS_PER_TILE, ROWS_PER_TILE)])


_sc_call = pl.kernel(
    _sc_body,
    out_type=jax.ShapeDtypeStruct((NC, N_PAD, D), jnp.float32),
    mesh=plsc.VectorSubcoreMesh(core_axis_name="c", subcore_axis_name="s"),
    scratch_types=[
        pltpu.VMEM_SHARED((N_PAD, D), jnp.float32),
        pltpu.VMEM((EDGES_PER_W,), jnp.int32),
        pltpu.VMEM((BLK,), jnp.int32),
        pltpu.VMEM((BLK,), jnp.int32),
        pltpu.VMEM((BLK,), jnp.int32),
        pltpu.VMEM((BLK,), jnp.float32),
        pltpu.VMEM((BLK,), jnp.float32),
        pltpu.VMEM((BLK,), jnp.float32),
        pltpu.VMEM((BLK, D), jnp.float32),
        pltpu.VMEM((BLK, D), jnp.float32),
        pltpu.VMEM((BLK, D), jnp.float32),
        pltpu.SemaphoreType.DMA,
        pltpu.SemaphoreType.DMA,
        pltpu.SemaphoreType.DMA,
        pltpu.SemaphoreType.DMA,
        pltpu.SemaphoreType.DMA,
        pltpu.SemaphoreType.DMA,
    ],
    name="gcn_spmm_sc",
)

_CBLK = 2000


def _combine_body(p_ref, q_ref, o_ref):
    o_ref[...] = p_ref[0] + q_ref[0]


_combine = pl.pallas_call(
    _combine_body,
    grid=(N_NODES // _CBLK,),
    in_specs=[
        pl.BlockSpec((1, _CBLK, D), lambda i: (0, i, 0)),
        pl.BlockSpec((1, _CBLK, D), lambda i: (1, i, 0)),
    ],
    out_specs=pl.BlockSpec((_CBLK, D), lambda i: (i, 0)),
    out_shape=jax.ShapeDtypeStruct((N_NODES, D), jnp.float32),
)


def kernel(input, edge_index, edge_weight):
    src = edge_index[0]
    dst = edge_index[1]
    partials = _sc_call(input, src, dst, edge_weight)
    return _combine(partials, partials)
